# baseline (speedup 1.0000x reference)
import jax
import jax.numpy as jnp
from jax import lax
from jax.experimental import pallas as pl
from jax.experimental.pallas import tpu as pltpu
from jax.experimental.pallas import tpu_sc as plsc

BATCH = 4
SEQ = 2048
D = 768
TOKENS = BATCH * SEQ
NUM_WORKERS = 32
S_PER_W = SEQ // NUM_WORKERS
SCH = 8
ROWS = BATCH * SCH
NUM_CHUNKS = S_PER_W // SCH
NBUF = 3
AHEAD = 1

_mesh = plsc.VectorSubcoreMesh(core_axis_name="c", subcore_axis_name="s")

_scratch = (
    [pltpu.VMEM((S_PER_W * BATCH,), jnp.int32)]
    + [pltpu.VMEM((S_PER_W, D), jnp.float32)]
    + [pltpu.VMEM((NBUF * ROWS, D), jnp.float32)]
    + [pltpu.SemaphoreType.DMA((NBUF,))]
    + [pltpu.SemaphoreType.DMA((NBUF,))]
    + [pltpu.SemaphoreType.DMA]
    + [pltpu.SemaphoreType.DMA]
)


@pl.kernel(
    out_type=jax.ShapeDtypeStruct((TOKENS, D), jnp.float32),
    mesh=_mesh,
    scratch_types=_scratch,
)
def _embed(ids_hbm, table_hbm, pos_hbm, out_hbm, idx_v, pbuf, ring,
           gsem, wsem, isem, psem):
    wid = lax.axis_index("s") * 2 + lax.axis_index("c")
    s_base = wid * S_PER_W

    id_copies = [
        pltpu.async_copy(
            ids_hbm.at[pl.ds(b * SEQ + s_base, S_PER_W)],
            idx_v.at[pl.ds(b * S_PER_W, S_PER_W)], isem)
        for b in range(BATCH)
    ]
    pos_copy = pltpu.async_copy(pos_hbm.at[pl.ds(s_base, S_PER_W)], pbuf, psem)
    for c in id_copies:
        c.wait()

    def gather_copies(j, slot):
        return [
            pltpu.make_async_copy(
                table_hbm.at[idx_v.at[pl.ds(b * S_PER_W + j * SCH, SCH)]],
                ring.at[pl.ds(slot * ROWS + b * SCH, SCH)], gsem.at[slot],
            )
            for b in range(BATCH)
        ]

    def write_copies(j, slot):
        return [
            pltpu.make_async_copy(
                ring.at[pl.ds(slot * ROWS + b * SCH, SCH)],
                out_hbm.at[pl.ds(b * SEQ + s_base + j * SCH, SCH)],
                wsem.at[slot],
            )
            for b in range(BATCH)
        ]

    for j in range(AHEAD):
        for c in gather_copies(j, j % NBUF):
            c.start()
    pos_copy.wait()

    def chunk_step(j, _):
        slot = lax.rem(j, NBUF)
        for c in gather_copies(j, slot):
            c.wait()

        def add_srow(r, _):
            prow = j * SCH + r
            for k in range(D // 16):
                sl = pl.ds(k * 16, 16)
                pvec = pbuf[prow, sl]
                for b in range(BATCH):
                    plsc.addupdate(ring.at[slot * ROWS + b * SCH + r, sl], pvec)
            return 0

        lax.fori_loop(0, SCH, add_srow, 0, unroll=False)

        nxt = j + AHEAD
        nslot = lax.rem(nxt, NBUF)

        @pl.when(nxt < NUM_CHUNKS)
        def _():
            @pl.when(nxt >= NBUF)
            def _():
                for c in write_copies(nxt - NBUF, nslot):
                    c.wait()
            for c in gather_copies(nxt, nslot):
                c.start()

        for c in write_copies(j, slot):
            c.start()
        return 0

    lax.fori_loop(0, NUM_CHUNKS, chunk_step, 0, unroll=False)

    for j in range(NUM_CHUNKS - NBUF, NUM_CHUNKS):
        for c in write_copies(j, j % NBUF):
            c.wait()


def kernel(token_ids, token_table, pos_table):
    out = _embed(token_ids.reshape(TOKENS), token_table, pos_table)
    return out.reshape(BATCH, SEQ, D)


# speedup vs baseline: 1.2827x; 1.2827x over previous
import jax
import jax.numpy as jnp
from jax import lax
from jax.experimental import pallas as pl
from jax.experimental.pallas import tpu as pltpu
from jax.experimental.pallas import tpu_sc as plsc

BATCH = 4
SEQ = 2048
D = 768
TOKENS = BATCH * SEQ
NUM_WORKERS = 32
S_PER_W = SEQ // NUM_WORKERS
SCH = 8
ROWS = BATCH * SCH
NUM_CHUNKS = S_PER_W // SCH
NBUF = 3
AHEAD = 2

_mesh = plsc.VectorSubcoreMesh(core_axis_name="c", subcore_axis_name="s")

_scratch = (
    [pltpu.VMEM((S_PER_W * BATCH,), jnp.int32)]
    + [pltpu.VMEM((S_PER_W, D), jnp.float32)]
    + [pltpu.VMEM((NBUF * ROWS, D), jnp.float32)]
    + [pltpu.SemaphoreType.DMA((NBUF,))]
    + [pltpu.SemaphoreType.DMA((NBUF,))]
    + [pltpu.SemaphoreType.DMA]
    + [pltpu.SemaphoreType.DMA]
)


@pl.kernel(
    out_type=jax.ShapeDtypeStruct((TOKENS, D), jnp.float32),
    mesh=_mesh,
    scratch_types=_scratch,
)
def _embed(ids_hbm, table_hbm, pos_hbm, out_hbm, idx_v, pbuf, ring,
           gsem, wsem, isem, psem):
    wid = lax.axis_index("s") * 2 + lax.axis_index("c")
    s_base = wid * S_PER_W

    id_copies = [
        pltpu.async_copy(
            ids_hbm.at[pl.ds(b * SEQ + s_base, S_PER_W)],
            idx_v.at[pl.ds(b * S_PER_W, S_PER_W)], isem)
        for b in range(BATCH)
    ]
    pos_copy = pltpu.async_copy(pos_hbm.at[pl.ds(s_base, S_PER_W)], pbuf, psem)
    for c in id_copies:
        c.wait()

    def gather_copies(j, slot):
        return [
            pltpu.make_async_copy(
                table_hbm.at[idx_v.at[pl.ds(b * S_PER_W + j * SCH, SCH)]],
                ring.at[pl.ds(slot * ROWS + b * SCH, SCH)], gsem.at[slot],
            )
            for b in range(BATCH)
        ]

    def write_copies(j, slot):
        return [
            pltpu.make_async_copy(
                ring.at[pl.ds(slot * ROWS + b * SCH, SCH)],
                out_hbm.at[pl.ds(b * SEQ + s_base + j * SCH, SCH)],
                wsem.at[slot],
            )
            for b in range(BATCH)
        ]

    for j in range(AHEAD):
        for c in gather_copies(j, j % NBUF):
            c.start()
    pos_copy.wait()

    def chunk_step(j, _):
        slot = lax.rem(j, NBUF)
        for c in gather_copies(j, slot):
            c.wait()

        def add_srow(r, _):
            prow = j * SCH + r
            for k in range(D // 16):
                sl = pl.ds(k * 16, 16)
                pvec = pbuf[prow, sl]
                for b in range(BATCH):
                    plsc.addupdate(ring.at[slot * ROWS + b * SCH + r, sl], pvec)
            return 0

        lax.fori_loop(0, SCH, add_srow, 0, unroll=False)

        nxt = j + AHEAD
        nslot = lax.rem(nxt, NBUF)

        @pl.when(nxt < NUM_CHUNKS)
        def _():
            @pl.when(nxt >= NBUF)
            def _():
                for c in write_copies(nxt - NBUF, nslot):
                    c.wait()
            for c in gather_copies(nxt, nslot):
                c.start()

        for c in write_copies(j, slot):
            c.start()
        return 0

    lax.fori_loop(0, NUM_CHUNKS, chunk_step, 0, unroll=False)

    for j in range(NUM_CHUNKS - NBUF, NUM_CHUNKS):
        for c in write_copies(j, j % NBUF):
            c.wait()


def kernel(token_ids, token_table, pos_table):
    out = _embed(token_ids.reshape(TOKENS), token_table, pos_table)
    return out.reshape(BATCH, SEQ, D)
